# Initial kernel scaffold; baseline (speedup 1.0000x reference)
import jax
import jax.numpy as jnp
from jax.experimental import pallas as pl


def kernel(local_preds, idx, origin_fea, A_hat, Wq_w, Wq_b, Wk_w, Wk_b, lin1_w, lin2_w, gumbel):
    raise NotImplementedError("write your pallas kernel here")


# speedup vs baseline: 2.3831x; 2.3831x over previous
import functools

import jax
import jax.numpy as jnp
from jax.experimental import pallas as pl
from jax.experimental.pallas import tpu as pltpu

N = 2048
H = 8
DH = 64
F = 512
NIT = 8
TOPK = 10
BLK = 256
NNCLS = 64


def _proj_body(fea_ref, wq_ref, bq_ref, wk_ref, bk_ref, qs_ref, k_ref):
    fea = fea_ref[...]
    dn = (((1,), (1,)), ((), ()))
    q = jax.lax.dot_general(fea, wq_ref[...], dn,
                            preferred_element_type=jnp.float32) + bq_ref[...]
    k = jax.lax.dot_general(fea, wk_ref[...], dn,
                            preferred_element_type=jnp.float32) + bk_ref[...]
    s = jnp.sqrt(jnp.sum(q * q) * jnp.sum(k * k))
    ks_sum = jnp.sum(k, axis=0, keepdims=True)
    scales = []
    for h in range(H):
        sl = slice(h * DH, (h + 1) * DH)
        dh = jnp.sum(q[:, sl] * ks_sum[:, sl], axis=1, keepdims=True)
        c = 1.0 / (H * (dh + N * s))
        scales.append(jnp.broadcast_to(c, (N, DH)))
    qs_ref[...] = q * jnp.concatenate(scales, axis=1)
    k_ref[...] = k


def _attn_body(qs_ref, kf_ref, gum_ref, attn_ref):
    logits = jax.lax.dot_general(qs_ref[...], kf_ref[...],
                                 (((1,), (1,)), ((), ())),
                                 preferred_element_type=jnp.float32)
    logits = logits + gum_ref[...]
    m = jnp.max(logits, axis=1, keepdims=True)
    e = jnp.exp(logits - m)
    gs = e / jnp.sum(e, axis=1, keepdims=True)
    colid = jax.lax.broadcasted_iota(jnp.int32, gs.shape, 1)
    work = gs
    sel = jnp.zeros(gs.shape, dtype=jnp.bool_)
    for _ in range(TOPK):
        mx = jnp.max(work, axis=1, keepdims=True)
        cand = jnp.where(work == mx, colid, N)
        chosen = colid == jnp.min(cand, axis=1, keepdims=True)
        sel = jnp.logical_or(sel, chosen)
        work = jnp.where(chosen, -jnp.inf, work)
    attn_ref[...] = jnp.where(sel, gs, 0.0)


def _square_body(ablk_ref, afull_ref, out_ref):
    out_ref[...] = jax.lax.dot_general(ablk_ref[...], afull_ref[...],
                                       (((1,), (0,)), ((), ())),
                                       preferred_element_type=jnp.float32)


def _horner_body(a2_ref, ah_ref, preds_ref, b2_ref, b1_ref, out_ref):
    dn = (((1,), (0,)), ((), ()))
    a2 = a2_ref[...]
    tmp = preds_ref[...]
    acc = tmp * b2_ref[0, 0]
    for i in range(1, NIT):
        tmp = jax.lax.dot_general(a2, tmp, dn, preferred_element_type=jnp.float32)
        acc = acc + tmp * b2_ref[0, i]
    ah = ah_ref[...]
    tmp = acc
    acc = tmp * b1_ref[0, 0]
    for i in range(1, NIT):
        tmp = jax.lax.dot_general(ah, tmp, dn, preferred_element_type=jnp.float32)
        acc = acc + tmp * b1_ref[0, i]
    out_ref[...] = acc


def kernel(local_preds, idx, origin_fea, A_hat, Wq_w, Wq_b, Wk_w, Wk_b,
           lin1_w, lin2_w, gumbel):
    f32 = jnp.float32
    bq = Wq_b.reshape(1, F)
    bk = Wk_b.reshape(1, F)

    qs, k = pl.pallas_call(
        _proj_body,
        out_shape=[jax.ShapeDtypeStruct((N, F), f32),
                   jax.ShapeDtypeStruct((N, F), f32)],
    )(origin_fea, Wq_w, bq, Wk_w, bk)

    nblk = N // BLK
    attn = pl.pallas_call(
        _attn_body,
        grid=(nblk,),
        in_specs=[pl.BlockSpec((BLK, F), lambda i: (i, 0)),
                  pl.BlockSpec((N, F), lambda i: (0, 0)),
                  pl.BlockSpec((BLK, N), lambda i: (i, 0))],
        out_specs=pl.BlockSpec((BLK, N), lambda i: (i, 0)),
        out_shape=jax.ShapeDtypeStruct((N, N), f32),
    )(qs, k, gumbel)

    attn2 = pl.pallas_call(
        _square_body,
        grid=(nblk,),
        in_specs=[pl.BlockSpec((BLK, N), lambda i: (i, 0)),
                  pl.BlockSpec((N, N), lambda i: (0, 0))],
        out_specs=pl.BlockSpec((BLK, N), lambda i: (i, 0)),
        out_shape=jax.ShapeDtypeStruct((N, N), f32),
    )(attn, attn)

    out = pl.pallas_call(
        _horner_body,
        in_specs=[pl.BlockSpec(memory_space=pltpu.MemorySpace.VMEM),
                  pl.BlockSpec(memory_space=pltpu.MemorySpace.VMEM),
                  pl.BlockSpec(memory_space=pltpu.MemorySpace.VMEM),
                  pl.BlockSpec(memory_space=pltpu.MemorySpace.SMEM),
                  pl.BlockSpec(memory_space=pltpu.MemorySpace.SMEM)],
        out_shape=jax.ShapeDtypeStruct((N, NNCLS), f32),
    )(attn2, A_hat, local_preds, lin2_w, lin1_w)
    return out


# speedup vs baseline: 2.3913x; 1.0035x over previous
import functools

import jax
import jax.numpy as jnp
from jax.experimental import pallas as pl
from jax.experimental.pallas import tpu as pltpu

N = 2048
H = 8
DH = 64
F = 512
NIT = 8
TOPK = 10
BLK = 256
NNCLS = 64


def _proj_body(fea_ref, wq_ref, bq_ref, wk_ref, bk_ref, qs_ref, k_ref):
    fea = fea_ref[...]
    dn = (((1,), (1,)), ((), ()))
    q = jax.lax.dot_general(fea, wq_ref[...], dn,
                            preferred_element_type=jnp.float32) + bq_ref[...]
    k = jax.lax.dot_general(fea, wk_ref[...], dn,
                            preferred_element_type=jnp.float32) + bk_ref[...]
    s = jnp.sqrt(jnp.sum(q * q) * jnp.sum(k * k))
    ks_sum = jnp.sum(k, axis=0, keepdims=True)
    scales = []
    for h in range(H):
        sl = slice(h * DH, (h + 1) * DH)
        dh = jnp.sum(q[:, sl] * ks_sum[:, sl], axis=1, keepdims=True)
        c = 1.0 / (H * (dh + N * s))
        scales.append(jnp.broadcast_to(c, (N, DH)))
    qs_ref[...] = q * jnp.concatenate(scales, axis=1)
    k_ref[...] = k


def _attn_body(qs_ref, kf_ref, gum_ref, attn_ref):
    logits = jax.lax.dot_general(qs_ref[...], kf_ref[...],
                                 (((1,), (1,)), ((), ())),
                                 preferred_element_type=jnp.float32)
    logits = logits + gum_ref[...]
    m = jnp.max(logits, axis=1, keepdims=True)
    e = jnp.exp(logits - m)
    gs = e / jnp.sum(e, axis=1, keepdims=True)
    colid = jax.lax.broadcasted_iota(jnp.int32, gs.shape, 1)
    work = gs
    sel = jnp.zeros(gs.shape, dtype=jnp.bool_)
    for _ in range(TOPK):
        mx = jnp.max(work, axis=1, keepdims=True)
        cand = jnp.where(work == mx, colid, N)
        chosen = colid == jnp.min(cand, axis=1, keepdims=True)
        sel = jnp.logical_or(sel, chosen)
        work = jnp.where(chosen, -jnp.inf, work)
    attn_ref[...] = jnp.where(sel, gs, 0.0)


def _square_body(ablk_ref, afull_ref, out_ref):
    out_ref[...] = jax.lax.dot_general(ablk_ref[...], afull_ref[...],
                                       (((1,), (0,)), ((), ())),
                                       preferred_element_type=jnp.float32)


def _horner_body(a_ref, ah_ref, preds_ref, b2_ref, b1_ref, out_ref):
    dn = (((1,), (0,)), ((), ()))
    a = a_ref[...]
    tmp = preds_ref[...]
    acc = tmp * b2_ref[0, 0]
    for i in range(1, NIT):
        tmp = jax.lax.dot_general(a, tmp, dn, preferred_element_type=jnp.float32)
        tmp = jax.lax.dot_general(a, tmp, dn, preferred_element_type=jnp.float32)
        acc = acc + tmp * b2_ref[0, i]
    ah = ah_ref[...]
    tmp = acc
    acc = tmp * b1_ref[0, 0]
    for i in range(1, NIT):
        tmp = jax.lax.dot_general(ah, tmp, dn, preferred_element_type=jnp.float32)
        acc = acc + tmp * b1_ref[0, i]
    out_ref[...] = acc


def kernel(local_preds, idx, origin_fea, A_hat, Wq_w, Wq_b, Wk_w, Wk_b,
           lin1_w, lin2_w, gumbel):
    f32 = jnp.float32
    bq = Wq_b.reshape(1, F)
    bk = Wk_b.reshape(1, F)

    qs, k = pl.pallas_call(
        _proj_body,
        out_shape=[jax.ShapeDtypeStruct((N, F), f32),
                   jax.ShapeDtypeStruct((N, F), f32)],
    )(origin_fea, Wq_w, bq, Wk_w, bk)

    nblk = N // BLK
    attn = pl.pallas_call(
        _attn_body,
        grid=(nblk,),
        in_specs=[pl.BlockSpec((BLK, F), lambda i: (i, 0)),
                  pl.BlockSpec((N, F), lambda i: (0, 0)),
                  pl.BlockSpec((BLK, N), lambda i: (i, 0))],
        out_specs=pl.BlockSpec((BLK, N), lambda i: (i, 0)),
        out_shape=jax.ShapeDtypeStruct((N, N), f32),
    )(qs, k, gumbel)

    out = pl.pallas_call(
        _horner_body,
        in_specs=[pl.BlockSpec(memory_space=pltpu.MemorySpace.VMEM),
                  pl.BlockSpec(memory_space=pltpu.MemorySpace.VMEM),
                  pl.BlockSpec(memory_space=pltpu.MemorySpace.VMEM),
                  pl.BlockSpec(memory_space=pltpu.MemorySpace.SMEM),
                  pl.BlockSpec(memory_space=pltpu.MemorySpace.SMEM)],
        out_shape=jax.ShapeDtypeStruct((N, NNCLS), f32),
    )(attn, A_hat, local_preds, lin2_w, lin1_w)
    return out
